# baseline (speedup 1.0000x reference)
import jax
import jax.numpy as jnp
from jax import lax
from jax.experimental import pallas as pl
from jax.experimental.pallas import tpu as pltpu

N = 256
D = 128
HEADS = 4
DH = D // HEADS
TD = 64
NEG_INF = float("-inf")


def _gat_layer(x, wl, wr, att_ref, att06_ref, bias, adj):
    xl = lax.dot_general(x, wl, (((1,), (0,)), ((), ())),
                         preferred_element_type=jnp.float32)
    xr = lax.dot_general(x, wr, (((1,), (0,)), ((), ())),
                         preferred_element_type=jnp.float32)
    xlt = xl.T
    xrb = xr.astype(jnp.bfloat16)
    xltb = xlt.astype(jnp.bfloat16)
    outs = []
    for h in range(HEADS):
        xl_h = xl[:, h * DH:(h + 1) * DH]
        xr_h = xr[:, h * DH:(h + 1) * DH]
        a06 = att06_ref[h:h + 1, :]
        sl_row = lax.dot_general(a06, xl_h, (((1,), (1,)), ((), ())),
                                 preferred_element_type=jnp.float32)
        sr_col = lax.dot_general(xr_h, a06, (((1,), (1,)), ((), ())),
                                 preferred_element_type=jnp.float32)
        acc0 = jnp.zeros((N, N), jnp.bfloat16)
        acc1 = jnp.zeros((N, N), jnp.bfloat16)
        for k in range(0, DH, 2):
            c = h * DH + k
            col = xrb[:, c:c + 1]
            row = xltb[c:c + 1, :]
            s_k = (att_ref[h, k] * 0.4).astype(jnp.bfloat16)
            acc0 = acc0 + jnp.abs(col + row) * s_k
            col1 = xrb[:, c + 1:c + 2]
            row1 = xltb[c + 1:c + 2, :]
            s_k1 = (att_ref[h, k + 1] * 0.4).astype(jnp.bfloat16)
            acc1 = acc1 + jnp.abs(col1 + row1) * s_k1
        e0 = (sr_col + sl_row) + (acc0 + acc1).astype(jnp.float32)
        e = jnp.where(adj != 0.0, e0, NEG_INF)
        m = jnp.max(e, axis=1, keepdims=True)
        m = jnp.where(jnp.isfinite(m), m, 0.0)
        ex = jnp.exp(e - m)
        denom = jnp.sum(ex, axis=1, keepdims=True)
        alpha = ex / (denom + 1e-16)
        outs.append(lax.dot_general(
            alpha, xl[:, h * DH:(h + 1) * DH],
            (((1,), (0,)), ((), ())),
            preferred_element_type=jnp.float32))
    return jnp.concatenate(outs, axis=1) + bias


def _body(x_ref, emb_ref, wl1_ref, wr1_ref, b1_ref, wl2_ref, wr2_ref,
          b2_ref, att1v_ref, att2v_ref, att1_ref, att2_ref, out_ref,
          adj_scr):
    @pl.when(pl.program_id(0) == 0)
    def _():
        emb = emb_ref[...]
        sq = jnp.sum(emb * emb, axis=1, keepdims=True)
        nrm = jnp.maximum(jnp.sqrt(sq), 1e-12)
        ne = emb / nrm
        adj_scr[...] = lax.dot_general(ne, ne, (((1,), (1,)), ((), ())),
                                       preferred_element_type=jnp.float32)

    x = x_ref[0]
    adj = adj_scr[...]
    h1 = _gat_layer(x, wl1_ref[...], wr1_ref[...], att1_ref, att1v_ref[...],
                    b1_ref[...], adj)
    h1 = jnp.maximum(h1, 0.0)
    out_ref[0] = _gat_layer(h1, wl2_ref[...], wr2_ref[...], att2_ref,
                            att2v_ref[...], b2_ref[...], adj)


def kernel(x, embedding, Wl1, Wr1, att1, b1, Wl2, Wr2, att2, b2):
    batch = x.shape[0]
    full = lambda shape: pl.BlockSpec(shape, lambda b: (0,) * len(shape))
    out = pl.pallas_call(
        _body,
        grid=(batch,),
        in_specs=[
            pl.BlockSpec((1, N, D), lambda b: (b, 0, 0)),
            full((N, D)),
            full((D, D)),
            full((D, D)),
            full((1, D)),
            full((D, D)),
            full((D, D)),
            full((1, D)),
            full((HEADS, DH)),
            full((HEADS, DH)),
            pl.BlockSpec(memory_space=pltpu.SMEM),
            pl.BlockSpec(memory_space=pltpu.SMEM),
        ],
        out_specs=pl.BlockSpec((1, N, D), lambda b: (b, 0, 0)),
        out_shape=jax.ShapeDtypeStruct((batch, N, D), jnp.float32),
        scratch_shapes=[pltpu.VMEM((N, N), jnp.float32)],
    )(x, embedding, Wl1, Wr1, b1.reshape(1, D), Wl2, Wr2,
      b2.reshape(1, D), 0.6 * att1, 0.6 * att2, att1, att2)
    return out


# speedup vs baseline: 1.0496x; 1.0496x over previous
import jax
import jax.numpy as jnp
from jax import lax
from jax.experimental import pallas as pl
from jax.experimental.pallas import tpu as pltpu

N = 256
D = 128
HEADS = 4
DH = D // HEADS
TD = 64
NEG_INF = float("-inf")


def _gat_layer(x, wl, wr, att_ref, att06_ref, bias, adj):
    xl = lax.dot_general(x, wl, (((1,), (0,)), ((), ())),
                         preferred_element_type=jnp.float32)
    xr = lax.dot_general(x, wr, (((1,), (0,)), ((), ())),
                         preferred_element_type=jnp.float32)
    xlt = xl.T
    xrb = xr.astype(jnp.bfloat16)
    xltb = xlt.astype(jnp.bfloat16)
    outs = []
    for h in range(HEADS):
        xl_h = xl[:, h * DH:(h + 1) * DH]
        xr_h = xr[:, h * DH:(h + 1) * DH]
        a06 = att06_ref[h:h + 1, :]
        sl_row = lax.dot_general(a06, xl_h, (((1,), (1,)), ((), ())),
                                 preferred_element_type=jnp.float32)
        sr_col = lax.dot_general(xr_h, a06, (((1,), (1,)), ((), ())),
                                 preferred_element_type=jnp.float32)
        acc0 = jnp.zeros((N, N), jnp.bfloat16)
        acc1 = jnp.zeros((N, N), jnp.bfloat16)
        for k in range(0, DH, 2):
            c = h * DH + k
            col = xrb[:, c:c + 1]
            row = xltb[c:c + 1, :]
            s_k = (att_ref[h, k] * 0.4).astype(jnp.bfloat16)
            acc0 = acc0 + jnp.abs(col + row) * s_k
            col1 = xrb[:, c + 1:c + 2]
            row1 = xltb[c + 1:c + 2, :]
            s_k1 = (att_ref[h, k + 1] * 0.4).astype(jnp.bfloat16)
            acc1 = acc1 + jnp.abs(col1 + row1) * s_k1
        e0 = (sr_col + sl_row) + (acc0 + acc1).astype(jnp.float32)
        e = jnp.where(adj != 0.0, e0, NEG_INF)
        m = jnp.max(e, axis=1, keepdims=True)
        m = jnp.where(jnp.isfinite(m), m, 0.0)
        ex = jnp.exp(e - m)
        denom = jnp.sum(ex, axis=1, keepdims=True)
        alpha = ex / (denom + 1e-16)
        outs.append(lax.dot_general(
            alpha, xl[:, h * DH:(h + 1) * DH],
            (((1,), (0,)), ((), ())),
            preferred_element_type=jnp.float32))
    return jnp.concatenate(outs, axis=1) + bias


def _body(x_ref, emb_ref, wl1_ref, wr1_ref, b1_ref, wl2_ref, wr2_ref,
          b2_ref, att1v_ref, att2v_ref, att1_ref, att2_ref, out_ref):
    emb = emb_ref[...]
    sq = jnp.sum(emb * emb, axis=1, keepdims=True)
    nrm = jnp.maximum(jnp.sqrt(sq), 1e-12)
    ne = emb / nrm
    adj = lax.dot_general(ne, ne, (((1,), (1,)), ((), ())),
                          preferred_element_type=jnp.float32)
    for i in range(2):
        x = x_ref[i]
        h1 = _gat_layer(x, wl1_ref[...], wr1_ref[...], att1_ref,
                        att1v_ref[...], b1_ref[...], adj)
        h1 = jnp.maximum(h1, 0.0)
        out_ref[i] = _gat_layer(h1, wl2_ref[...], wr2_ref[...], att2_ref,
                                att2v_ref[...], b2_ref[...], adj)


def kernel(x, embedding, Wl1, Wr1, att1, b1, Wl2, Wr2, att2, b2):
    batch = x.shape[0]
    full = lambda shape: pl.BlockSpec(shape, lambda b: (0,) * len(shape))
    out = pl.pallas_call(
        _body,
        grid=(batch // 2,),
        in_specs=[
            pl.BlockSpec((2, N, D), lambda b: (b, 0, 0)),
            full((N, D)),
            full((D, D)),
            full((D, D)),
            full((1, D)),
            full((D, D)),
            full((D, D)),
            full((1, D)),
            full((HEADS, DH)),
            full((HEADS, DH)),
            pl.BlockSpec(memory_space=pltpu.SMEM),
            pl.BlockSpec(memory_space=pltpu.SMEM),
        ],
        out_specs=pl.BlockSpec((2, N, D), lambda b: (b, 0, 0)),
        out_shape=jax.ShapeDtypeStruct((batch, N, D), jnp.float32),
    )(x, embedding, Wl1, Wr1, b1.reshape(1, D), Wl2, Wr2,
      b2.reshape(1, D), 0.6 * att1, 0.6 * att2, att1, att2)
    return out


# speedup vs baseline: 1.2158x; 1.1584x over previous
import jax
import jax.numpy as jnp
from jax import lax
from jax.experimental import pallas as pl
from jax.experimental.pallas import tpu as pltpu

N = 256
D = 128
HEADS = 4
DH = D // HEADS
TD = 64
NEG_INF = float("-inf")


def _gat_layer(x, wl, wr, att_ref, att06_ref, bias, adj):
    xl = lax.dot_general(x, wl, (((1,), (0,)), ((), ())),
                         preferred_element_type=jnp.float32)
    xr = lax.dot_general(x, wr, (((1,), (0,)), ((), ())),
                         preferred_element_type=jnp.float32)
    xlt = xl.T
    xrb = xr.astype(jnp.bfloat16)
    xltb = xlt.astype(jnp.bfloat16)
    outs = []
    for h in range(HEADS):
        xl_h = xl[:, h * DH:(h + 1) * DH]
        a06 = att06_ref[h:h + 1, :]
        sl_row = lax.dot_general(a06, xl_h, (((1,), (1,)), ((), ())),
                                 preferred_element_type=jnp.float32)
        acc0 = jnp.zeros((N, N), jnp.bfloat16)
        acc1 = jnp.zeros((N, N), jnp.bfloat16)
        for k in range(0, DH, 2):
            c = h * DH + k
            col = xrb[:, c:c + 1]
            row = xltb[c:c + 1, :]
            s_k = (att_ref[h, k] * 0.4).astype(jnp.bfloat16)
            acc0 = acc0 + jnp.abs(col + row) * s_k
            col1 = xrb[:, c + 1:c + 2]
            row1 = xltb[c + 1:c + 2, :]
            s_k1 = (att_ref[h, k + 1] * 0.4).astype(jnp.bfloat16)
            acc1 = acc1 + jnp.abs(col1 + row1) * s_k1
        e0 = sl_row + (acc0 + acc1).astype(jnp.float32)
        e = jnp.where(adj != 0.0, e0, NEG_INF)
        m = jnp.max(e, axis=1, keepdims=True)
        m = jnp.where(jnp.isfinite(m), m, 0.0)
        ex = jnp.exp(e - m)
        denom = jnp.sum(ex, axis=1, keepdims=True)
        alpha = ex / (denom + 1e-16)
        outs.append(lax.dot_general(
            alpha, xl[:, h * DH:(h + 1) * DH],
            (((1,), (0,)), ((), ())),
            preferred_element_type=jnp.float32))
    return jnp.concatenate(outs, axis=1) + bias


def _body(x_ref, emb_ref, wl1_ref, wr1_ref, b1_ref, wl2_ref, wr2_ref,
          b2_ref, att1v_ref, att2v_ref, att1_ref, att2_ref, out_ref):
    emb = emb_ref[...]
    sq = jnp.sum(emb * emb, axis=1, keepdims=True)
    nrm = jnp.maximum(jnp.sqrt(sq), 1e-12)
    ne = emb / nrm
    adj = lax.dot_general(ne, ne, (((1,), (1,)), ((), ())),
                          preferred_element_type=jnp.float32)
    for i in range(2):
        x = x_ref[i]
        h1 = _gat_layer(x, wl1_ref[...], wr1_ref[...], att1_ref,
                        att1v_ref[...], b1_ref[...], adj)
        h1 = jnp.maximum(h1, 0.0)
        out_ref[i] = _gat_layer(h1, wl2_ref[...], wr2_ref[...], att2_ref,
                                att2v_ref[...], b2_ref[...], adj)


def kernel(x, embedding, Wl1, Wr1, att1, b1, Wl2, Wr2, att2, b2):
    batch = x.shape[0]
    full = lambda shape: pl.BlockSpec(shape, lambda b: (0,) * len(shape))
    out = pl.pallas_call(
        _body,
        grid=(batch // 2,),
        in_specs=[
            pl.BlockSpec((2, N, D), lambda b: (b, 0, 0)),
            full((N, D)),
            full((D, D)),
            full((D, D)),
            full((1, D)),
            full((D, D)),
            full((D, D)),
            full((1, D)),
            full((HEADS, DH)),
            full((HEADS, DH)),
            pl.BlockSpec(memory_space=pltpu.SMEM),
            pl.BlockSpec(memory_space=pltpu.SMEM),
        ],
        out_specs=pl.BlockSpec((2, N, D), lambda b: (b, 0, 0)),
        out_shape=jax.ShapeDtypeStruct((batch, N, D), jnp.float32),
    )(x, embedding, Wl1, Wr1, b1.reshape(1, D), Wl2, Wr2,
      b2.reshape(1, D), 0.6 * att1, 0.6 * att2, att1, att2)
    return out


# speedup vs baseline: 1.2177x; 1.0015x over previous
import jax
import jax.numpy as jnp
from jax import lax
from jax.experimental import pallas as pl
from jax.experimental.pallas import tpu as pltpu

N = 256
D = 128
HEADS = 4
DH = D // HEADS
NEG_INF = float("-inf")


def _gat_layer(x, wl, wr, att_ref, att06_ref, bias, adj):
    xl = lax.dot_general(x, wl, (((1,), (0,)), ((), ())),
                         preferred_element_type=jnp.float32)
    xr = lax.dot_general(x, wr, (((1,), (0,)), ((), ())),
                         preferred_element_type=jnp.float32)
    xlt = xl.T
    xrb = xr.astype(jnp.bfloat16)
    xltb = xlt.astype(jnp.bfloat16)
    outs = []
    for h in range(HEADS):
        xl_h = xl[:, h * DH:(h + 1) * DH]
        a06 = att06_ref[h:h + 1, :]
        sl_row = lax.dot_general(a06, xl_h, (((1,), (1,)), ((), ())),
                                 preferred_element_type=jnp.float32)
        acc0 = jnp.zeros((N, N), jnp.bfloat16)
        acc1 = jnp.zeros((N, N), jnp.bfloat16)
        for k in range(0, DH, 2):
            c = h * DH + k
            col = xrb[:, c:c + 1]
            row = xltb[c:c + 1, :]
            s_k = (att_ref[h, k] * 0.4).astype(jnp.bfloat16)
            acc0 = acc0 + jnp.abs(col + row) * s_k
            col1 = xrb[:, c + 1:c + 2]
            row1 = xltb[c + 1:c + 2, :]
            s_k1 = (att_ref[h, k + 1] * 0.4).astype(jnp.bfloat16)
            acc1 = acc1 + jnp.abs(col1 + row1) * s_k1
        e0 = sl_row + (acc0 + acc1).astype(jnp.float32)
        e = jnp.where(adj != 0.0, e0, NEG_INF)
        m = jnp.max(e, axis=1, keepdims=True)
        m = jnp.where(jnp.isfinite(m), m, 0.0)
        ex = jnp.exp(e - m)
        denom = jnp.sum(ex, axis=1, keepdims=True)
        alpha = ex / (denom + 1e-16)
        outs.append(lax.dot_general(
            alpha, xl[:, h * DH:(h + 1) * DH],
            (((1,), (0,)), ((), ())),
            preferred_element_type=jnp.float32))
    return jnp.concatenate(outs, axis=1) + bias


def _body(x_ref, emb_ref, wl1_ref, wr1_ref, b1_ref, wl2_ref, wr2_ref,
          b2_ref, att1v_ref, att2v_ref, att1_ref, att2_ref, out_ref):
    emb = emb_ref[...]
    sq = jnp.sum(emb * emb, axis=1, keepdims=True)
    nrm = jnp.maximum(jnp.sqrt(sq), 1e-12)
    ne = emb / nrm
    adj = lax.dot_general(ne, ne, (((1,), (1,)), ((), ())),
                          preferred_element_type=jnp.float32)
    for i in range(2):
        x = x_ref[i]
        h1 = _gat_layer(x, wl1_ref[...], wr1_ref[...], att1_ref,
                        att1v_ref[...], b1_ref[...], adj)
        h1 = jnp.maximum(h1, 0.0)
        out_ref[i] = _gat_layer(h1, wl2_ref[...], wr2_ref[...], att2_ref,
                                att2v_ref[...], b2_ref[...], adj)


def kernel(x, embedding, Wl1, Wr1, att1, b1, Wl2, Wr2, att2, b2):
    batch = x.shape[0]
    full = lambda shape: pl.BlockSpec(shape, lambda b: (0,) * len(shape))
    out = pl.pallas_call(
        _body,
        grid=(batch // 2,),
        in_specs=[
            pl.BlockSpec((2, N, D), lambda b: (b, 0, 0)),
            full((N, D)),
            full((D, D)),
            full((D, D)),
            full((1, D)),
            full((D, D)),
            full((D, D)),
            full((1, D)),
            full((HEADS, DH)),
            full((HEADS, DH)),
            pl.BlockSpec(memory_space=pltpu.SMEM),
            pl.BlockSpec(memory_space=pltpu.SMEM),
        ],
        out_specs=pl.BlockSpec((2, N, D), lambda b: (b, 0, 0)),
        out_shape=jax.ShapeDtypeStruct((batch, N, D), jnp.float32),
    )(x, embedding, Wl1, Wr1, b1.reshape(1, D), Wl2, Wr2,
      b2.reshape(1, D), 0.6 * att1, 0.6 * att2, att1, att2)
    return out
